# Initial kernel scaffold; baseline (speedup 1.0000x reference)
import jax
import jax.numpy as jnp
from jax.experimental import pallas as pl


def kernel(x):
    raise NotImplementedError("write your pallas kernel here")


# speedup vs baseline: 2.3809x; 2.3809x over previous
import functools

import numpy as np
import jax
import jax.numpy as jnp
from jax import lax
from jax.experimental import pallas as pl
from jax.experimental.pallas import tpu as pltpu
from jax.experimental.pallas import tpu_sc as plsc

B, N, D = 32, 4096, 1024
KEEP = N // 2
ROWS = B * KEEP

NC, NS = 2, 16
NW = NC * NS
PER_W = ROWS // NW
CHUNK = 64
NCHUNK = PER_W // CHUNK


_R0 = (13, 15, 26, 6)
_R1 = (17, 29, 16, 24)


def _threefry2x32(k0, k1, x0, x1):
    x0 = np.asarray(x0, np.uint32).copy()
    x1 = np.asarray(x1, np.uint32).copy()
    ks0, ks1 = np.uint32(k0), np.uint32(k1)
    ks2 = np.uint32(ks0 ^ ks1 ^ np.uint32(0x1BD11BDA))
    with np.errstate(over="ignore"):
        x0 = (x0 + ks0).astype(np.uint32)
        x1 = (x1 + ks1).astype(np.uint32)
        sched = [(ks1, ks2), (ks2, ks0), (ks0, ks1), (ks1, ks2), (ks2, ks0)]
        for r in range(5):
            for rot in (_R0 if r % 2 == 0 else _R1):
                x0 = (x0 + x1).astype(np.uint32)
                x1 = ((x1 << np.uint32(rot)) |
                      (x1 >> np.uint32(32 - rot))).astype(np.uint32)
                x1 = (x1 ^ x0).astype(np.uint32)
            a, b = sched[r]
            x0 = (x0 + a).astype(np.uint32)
            x1 = (x1 + b + np.uint32(r + 1)).astype(np.uint32)
    return x0, x1


def _bits32(k0, k1, n):
    i = np.arange(n, dtype=np.uint64)
    c1 = (i >> np.uint64(32)).astype(np.uint32)
    c2 = (i & np.uint64(0xFFFFFFFF)).astype(np.uint32)
    b1, b2 = _threefry2x32(k0, k1, c1, c2)
    return b1 ^ b2


def _split2(k0, k1):
    i = np.arange(2, dtype=np.uint64)
    c1 = (i >> np.uint64(32)).astype(np.uint32)
    c2 = (i & np.uint64(0xFFFFFFFF)).astype(np.uint32)
    b1, b2 = _threefry2x32(k0, k1, c1, c2)
    return (b1[0], b2[0]), (b1[1], b2[1])


def _np_permutation(seed, n):
    key = (np.uint32(seed >> 32), np.uint32(seed & 0xFFFFFFFF))
    x = np.arange(n, dtype=np.int64)
    num_rounds = int(np.ceil(3 * np.log(max(1, n)) /
                             np.log(np.iinfo(np.uint32).max)))
    for _ in range(num_rounds):
        key, sub = _split2(*key)
        sort_keys = _bits32(sub[0], sub[1], n)
        x = x[np.argsort(sort_keys, kind="stable")]
    return x


def _flat_idx() -> np.ndarray:
    perm = _np_permutation(42, N)[:KEEP]
    flat = np.arange(B, dtype=np.int64)[:, None] * N + perm[None, :]
    return np.ascontiguousarray(flat.reshape(-1).astype(np.int32))


_IDX = _flat_idx()


_mesh = plsc.VectorSubcoreMesh(core_axis_name="c", subcore_axis_name="s")


@functools.partial(
    pl.kernel,
    mesh=_mesh,
    out_type=jax.ShapeDtypeStruct((ROWS, D), jnp.float32),
    scratch_types=[
        pltpu.VMEM((PER_W,), jnp.int32),
        pltpu.VMEM((CHUNK, D), jnp.float32),
        pltpu.SemaphoreType.DMA,
    ],
)
def _gather_rows(x_hbm, idx_hbm, out_hbm, idx_v, rows_v, sem):
    wid = lax.axis_index("s") * NC + lax.axis_index("c")
    base = wid * PER_W
    pltpu.sync_copy(idx_hbm.at[pl.ds(base, PER_W)], idx_v)

    def body(j, carry):
        off = pl.multiple_of(j * CHUNK, 8)
        pltpu.async_copy(x_hbm.at[idx_v.at[pl.ds(off, CHUNK)]], rows_v, sem).wait()
        pltpu.sync_copy(rows_v, out_hbm.at[pl.ds(base + off, CHUNK)])
        return carry

    lax.fori_loop(0, NCHUNK, body, 0)


def kernel(x):
    idx = jnp.asarray(_IDX)
    out = _gather_rows(x.reshape(B * N, D), idx)
    return out.reshape(B, KEEP, D)


# speedup vs baseline: 2.5606x; 1.0755x over previous
import functools

import numpy as np
import jax
import jax.numpy as jnp
from jax import lax
from jax.experimental import pallas as pl
from jax.experimental.pallas import tpu as pltpu
from jax.experimental.pallas import tpu_sc as plsc

B, N, D = 32, 4096, 1024
KEEP = N // 2
ROWS = B * KEEP

NC, NS = 2, 16
NW = NC * NS
PER_W = ROWS // NW
CHUNK = 32
NCHUNK = PER_W // CHUNK


_R0 = (13, 15, 26, 6)
_R1 = (17, 29, 16, 24)


def _threefry2x32(k0, k1, x0, x1):
    x0 = np.asarray(x0, np.uint32).copy()
    x1 = np.asarray(x1, np.uint32).copy()
    ks0, ks1 = np.uint32(k0), np.uint32(k1)
    ks2 = np.uint32(ks0 ^ ks1 ^ np.uint32(0x1BD11BDA))
    with np.errstate(over="ignore"):
        x0 = (x0 + ks0).astype(np.uint32)
        x1 = (x1 + ks1).astype(np.uint32)
        sched = [(ks1, ks2), (ks2, ks0), (ks0, ks1), (ks1, ks2), (ks2, ks0)]
        for r in range(5):
            for rot in (_R0 if r % 2 == 0 else _R1):
                x0 = (x0 + x1).astype(np.uint32)
                x1 = ((x1 << np.uint32(rot)) |
                      (x1 >> np.uint32(32 - rot))).astype(np.uint32)
                x1 = (x1 ^ x0).astype(np.uint32)
            a, b = sched[r]
            x0 = (x0 + a).astype(np.uint32)
            x1 = (x1 + b + np.uint32(r + 1)).astype(np.uint32)
    return x0, x1


def _bits32(k0, k1, n):
    i = np.arange(n, dtype=np.uint64)
    c1 = (i >> np.uint64(32)).astype(np.uint32)
    c2 = (i & np.uint64(0xFFFFFFFF)).astype(np.uint32)
    b1, b2 = _threefry2x32(k0, k1, c1, c2)
    return b1 ^ b2


def _split2(k0, k1):
    i = np.arange(2, dtype=np.uint64)
    c1 = (i >> np.uint64(32)).astype(np.uint32)
    c2 = (i & np.uint64(0xFFFFFFFF)).astype(np.uint32)
    b1, b2 = _threefry2x32(k0, k1, c1, c2)
    return (b1[0], b2[0]), (b1[1], b2[1])


def _np_permutation(seed, n):
    key = (np.uint32(seed >> 32), np.uint32(seed & 0xFFFFFFFF))
    x = np.arange(n, dtype=np.int64)
    num_rounds = int(np.ceil(3 * np.log(max(1, n)) /
                             np.log(np.iinfo(np.uint32).max)))
    for _ in range(num_rounds):
        key, sub = _split2(*key)
        sort_keys = _bits32(sub[0], sub[1], n)
        x = x[np.argsort(sort_keys, kind="stable")]
    return x


def _flat_idx() -> np.ndarray:
    perm = _np_permutation(42, N)[:KEEP]
    flat = np.arange(B, dtype=np.int64)[:, None] * N + perm[None, :]
    return np.ascontiguousarray(flat.reshape(-1).astype(np.int32))


_IDX = _flat_idx()


_mesh = plsc.VectorSubcoreMesh(core_axis_name="c", subcore_axis_name="s")


@functools.partial(
    pl.kernel,
    mesh=_mesh,
    out_type=jax.ShapeDtypeStruct((ROWS, D), jnp.float32),
    scratch_types=[
        pltpu.VMEM((PER_W,), jnp.int32),
        pltpu.VMEM((CHUNK, D), jnp.float32),
        pltpu.VMEM((CHUNK, D), jnp.float32),
        pltpu.SemaphoreType.DMA,
        pltpu.SemaphoreType.DMA,
        pltpu.SemaphoreType.DMA,
        pltpu.SemaphoreType.DMA,
    ],
)
def _gather_rows(x_hbm, idx_hbm, out_hbm, idx_v, buf_a, buf_b,
                 gs_a, gs_b, ss_a, ss_b):
    wid = lax.axis_index("s") * NC + lax.axis_index("c")
    base = wid * PER_W
    pltpu.sync_copy(idx_hbm.at[pl.ds(base, PER_W)], idx_v)

    buf = (buf_a, buf_b)
    gs = (gs_a, gs_b)
    ss = (ss_a, ss_b)

    def _gather(c, k):
        off = pl.multiple_of(c * CHUNK, 8)
        return pltpu.make_async_copy(
            x_hbm.at[idx_v.at[pl.ds(off, CHUNK)]], buf[k], gs[k])

    def _scatter(c, k):
        off = pl.multiple_of(base + c * CHUNK, 8)
        return pltpu.make_async_copy(buf[k], out_hbm.at[pl.ds(off, CHUNK)], ss[k])

    _gather(0, 0).start()
    _gather(1, 1).start()
    _gather(0, 0).wait()
    _scatter(0, 0).start()

    def body(j, carry):
        c0 = 2 * j + 1
        c1 = 2 * j + 2
        _scatter(c0 - 1, 0).wait()
        _gather(c0 + 1, 0).start()
        _gather(c0, 1).wait()
        _scatter(c0, 1).start()

        _scatter(c1 - 1, 1).wait()
        _gather(c1 + 1, 1).start()
        _gather(c1, 0).wait()
        _scatter(c1, 0).start()
        return carry

    lax.fori_loop(0, (NCHUNK - 2) // 2, body, 0)

    _scatter(NCHUNK - 2, 0).wait()
    _gather(NCHUNK - 1, 1).wait()
    _scatter(NCHUNK - 1, 1).start()
    _scatter(NCHUNK - 1, 1).wait()


def kernel(x):
    idx = jnp.asarray(_IDX)
    out = _gather_rows(x.reshape(B * N, D), idx)
    return out.reshape(B, KEEP, D)
